# baseline (speedup 1.0000x reference)
import math

import jax
import jax.numpy as jnp
import numpy as np
from jax import lax
from jax.experimental import pallas as pl
from jax.experimental.pallas import tpu as pltpu
from jax.experimental.pallas import tpu_sc as plsc

SEQ_LEN = 2048
DIM = 128
BATCH = 16
SCALE = math.sqrt(float(DIM))

NUM_CORES = 2
NUM_SUBCORES = 16
NW = NUM_CORES * NUM_SUBCORES
P_PER_W = SEQ_LEN // NW
B_PER_STEP = 4
N_STEPS = BATCH // B_PER_STEP
ROWS_PER_STEP = B_PER_STEP * P_PER_W
NBUF = 3
LANES = 16
VECS_PER_ROW = DIM // LANES


def _positional_encoding_np():
    n = 10000.0
    pos = np.arange(SEQ_LEN, dtype=np.float64)[:, None]
    i = np.arange(DIM // 2, dtype=np.float64)[None, :]
    denom = n ** (2.0 * i / DIM)
    enc = np.zeros((SEQ_LEN, DIM), dtype=np.float32)
    enc[:, 0::2] = np.sin(pos / denom).astype(np.float32)
    enc[:, 1::2] = np.cos(pos / denom).astype(np.float32)
    return enc


_POS_ENC = _positional_encoding_np()


def _embed_body(table_hbm, idx_hbm, pos_hbm, out_hbm,
                idx_v, pos_v, b0, b1, b2, isem, gs0, gs1, gs2, ws0, ws1, ws2):
    bufs = [b0, b1, b2]
    gsems = [gs0, gs1, gs2]
    wsems = [ws0, ws1, ws2]
    wid = lax.axis_index("s") * NUM_CORES + lax.axis_index("c")
    pbase = wid * P_PER_W

    idx_hs = [
        pltpu.async_copy(idx_hbm.at[b, pl.ds(pbase, P_PER_W)],
                         idx_v.at[pl.ds(b * P_PER_W, P_PER_W)], isem)
        for b in range(BATCH)
    ]
    pltpu.sync_copy(pos_hbm.at[pl.ds(pbase, P_PER_W)], pos_v)
    for h in idx_hs:
        h.wait()

    def start_gather(s):
        idx_slice = idx_v.at[pl.ds(s * ROWS_PER_STEP, ROWS_PER_STEP)]
        return pltpu.async_copy(table_hbm.at[idx_slice], bufs[s % NBUF],
                                gsems[s % NBUF])

    gather_h = {0: start_gather(0)}
    write_h = {}

    for s in range(N_STEPS):
        buf = bufs[s % NBUF]
        gather_h.pop(s).wait()
        if s + 1 < N_STEPS:
            for h in write_h.pop(s + 1 - NBUF, ()):
                h.wait()
            gather_h[s + 1] = start_gather(s + 1)
        for h in write_h.pop(s - NBUF, ()):
            h.wait()

        cur = buf

        @plsc.parallel_loop(0, P_PER_W, 1, unroll=2)
        def fma_pos(p):
            for j in range(VECS_PER_ROW):
                sl = pl.ds(j * LANES, LANES)
                pv = pos_v[p, sl]
                for bb in range(B_PER_STEP):
                    r = bb * P_PER_W + p
                    cur[r, sl] = cur[r, sl] * SCALE + pv

        hs = []
        for bb in range(B_PER_STEP):
            b = s * B_PER_STEP + bb
            hs.append(pltpu.async_copy(
                buf.at[pl.ds(bb * P_PER_W, P_PER_W)],
                out_hbm.at[b, pl.ds(pbase, P_PER_W)],
                wsems[s % NBUF]))
        write_h[s] = hs

    for hs in write_h.values():
        for h in hs:
            h.wait()


@jax.jit
def _embed(idx, table):
    pos_enc = jnp.asarray(_POS_ENC)
    mesh = plsc.VectorSubcoreMesh(
        core_axis_name="c", subcore_axis_name="s",
        num_cores=NUM_CORES, num_subcores=NUM_SUBCORES)
    fn = pl.kernel(
        _embed_body,
        out_type=jax.ShapeDtypeStruct((BATCH, SEQ_LEN, DIM), jnp.float32),
        mesh=mesh,
        scratch_types=[
            pltpu.VMEM((BATCH * P_PER_W,), jnp.int32),
            pltpu.VMEM((P_PER_W, DIM), jnp.float32),
            pltpu.VMEM((ROWS_PER_STEP, DIM), jnp.float32),
            pltpu.VMEM((ROWS_PER_STEP, DIM), jnp.float32),
            pltpu.VMEM((ROWS_PER_STEP, DIM), jnp.float32),
            pltpu.SemaphoreType.DMA,
            pltpu.SemaphoreType.DMA,
            pltpu.SemaphoreType.DMA,
            pltpu.SemaphoreType.DMA,
            pltpu.SemaphoreType.DMA,
            pltpu.SemaphoreType.DMA,
            pltpu.SemaphoreType.DMA,
        ],
    )
    return fn(table, idx, pos_enc)


def kernel(inputs, table):
    return _embed(inputs.astype(jnp.int32), table)


# speedup vs baseline: 1.0708x; 1.0708x over previous
import math

import jax
import jax.numpy as jnp
import numpy as np
from jax import lax
from jax.experimental import pallas as pl
from jax.experimental.pallas import tpu as pltpu
from jax.experimental.pallas import tpu_sc as plsc

SEQ_LEN = 2048
DIM = 128
BATCH = 16
SCALE = math.sqrt(float(DIM))

NUM_CORES = 2
NUM_SUBCORES = 16
NW = NUM_CORES * NUM_SUBCORES
P_PER_W = SEQ_LEN // NW
B_PER_STEP = 4
N_STEPS = BATCH // B_PER_STEP
ROWS_PER_STEP = B_PER_STEP * P_PER_W
NBUF = 3
LANES = 16
VECS_PER_ROW = DIM // LANES


def _positional_encoding_np():
    n = 10000.0
    pos = np.arange(SEQ_LEN, dtype=np.float64)[:, None]
    i = np.arange(DIM // 2, dtype=np.float64)[None, :]
    denom = n ** (2.0 * i / DIM)
    enc = np.zeros((SEQ_LEN, DIM), dtype=np.float32)
    enc[:, 0::2] = np.sin(pos / denom).astype(np.float32)
    enc[:, 1::2] = np.cos(pos / denom).astype(np.float32)
    return enc


_POS_ENC = _positional_encoding_np()


def _embed_body(table_hbm, idx_hbm, pos_hbm, out_hbm,
                idx_v, pos_v, b0, b1, b2, isem, gs0, gs1, gs2, ws0, ws1, ws2):
    bufs = [b0, b1, b2]
    gsems = [gs0, gs1, gs2]
    wsems = [ws0, ws1, ws2]
    wid = lax.axis_index("s") * NUM_CORES + lax.axis_index("c")
    pbase = wid * P_PER_W

    idx_hs = [
        pltpu.async_copy(idx_hbm.at[b, pl.ds(pbase, P_PER_W)],
                         idx_v.at[pl.ds(b * P_PER_W, P_PER_W)], isem)
        for b in range(BATCH)
    ]
    pltpu.sync_copy(pos_hbm.at[pl.ds(pbase, P_PER_W)], pos_v)
    for h in idx_hs:
        h.wait()

    def start_gather(s):
        idx_slice = idx_v.at[pl.ds(s * ROWS_PER_STEP, ROWS_PER_STEP)]
        return pltpu.async_copy(table_hbm.at[idx_slice], bufs[s % NBUF],
                                gsems[s % NBUF])

    gather_h = {0: start_gather(0)}
    write_h = {}

    for s in range(N_STEPS):
        buf = bufs[s % NBUF]
        gather_h.pop(s).wait()
        if s + 1 < N_STEPS:
            for h in write_h.pop(s + 1 - NBUF, ()):
                h.wait()
            gather_h[s + 1] = start_gather(s + 1)
        for h in write_h.pop(s - NBUF, ()):
            h.wait()

        cur = buf

        hs = []
        for bb in range(B_PER_STEP):
            b = s * B_PER_STEP + bb
            hs.append(pltpu.async_copy(
                buf.at[pl.ds(bb * P_PER_W, P_PER_W)],
                out_hbm.at[b, pl.ds(pbase, P_PER_W)],
                wsems[s % NBUF]))
        write_h[s] = hs

    for hs in write_h.values():
        for h in hs:
            h.wait()


@jax.jit
def _embed(idx, table):
    pos_enc = jnp.asarray(_POS_ENC)
    mesh = plsc.VectorSubcoreMesh(
        core_axis_name="c", subcore_axis_name="s",
        num_cores=NUM_CORES, num_subcores=NUM_SUBCORES)
    fn = pl.kernel(
        _embed_body,
        out_type=jax.ShapeDtypeStruct((BATCH, SEQ_LEN, DIM), jnp.float32),
        mesh=mesh,
        scratch_types=[
            pltpu.VMEM((BATCH * P_PER_W,), jnp.int32),
            pltpu.VMEM((P_PER_W, DIM), jnp.float32),
            pltpu.VMEM((ROWS_PER_STEP, DIM), jnp.float32),
            pltpu.VMEM((ROWS_PER_STEP, DIM), jnp.float32),
            pltpu.VMEM((ROWS_PER_STEP, DIM), jnp.float32),
            pltpu.SemaphoreType.DMA,
            pltpu.SemaphoreType.DMA,
            pltpu.SemaphoreType.DMA,
            pltpu.SemaphoreType.DMA,
            pltpu.SemaphoreType.DMA,
            pltpu.SemaphoreType.DMA,
            pltpu.SemaphoreType.DMA,
        ],
    )
    return fn(table, idx, pos_enc)


def kernel(inputs, table):
    return _embed(inputs.astype(jnp.int32), table)


# speedup vs baseline: 1.1125x; 1.0389x over previous
import math

import jax
import jax.numpy as jnp
import numpy as np
from jax import lax
from jax.experimental import pallas as pl
from jax.experimental.pallas import tpu as pltpu
from jax.experimental.pallas import tpu_sc as plsc

SEQ_LEN = 2048
DIM = 128
BATCH = 16
SCALE = math.sqrt(float(DIM))

NUM_CORES = 2
NUM_SUBCORES = 16
NW = NUM_CORES * NUM_SUBCORES
P_PER_W = SEQ_LEN // NW
B_PER_STEP = 4
N_STEPS = BATCH // B_PER_STEP
ROWS_PER_STEP = B_PER_STEP * P_PER_W
NBUF = 3
LANES = 16
VECS_PER_ROW = DIM // LANES


def _positional_encoding_np():
    n = 10000.0
    pos = np.arange(SEQ_LEN, dtype=np.float64)[:, None]
    i = np.arange(DIM // 2, dtype=np.float64)[None, :]
    denom = n ** (2.0 * i / DIM)
    enc = np.zeros((SEQ_LEN, DIM), dtype=np.float32)
    enc[:, 0::2] = np.sin(pos / denom).astype(np.float32)
    enc[:, 1::2] = np.cos(pos / denom).astype(np.float32)
    return enc.reshape(-1)


_POS_ENC = _positional_encoding_np()


def _embed_body(table_hbm, idx_hbm, pos_hbm, out_hbm,
                idx_v, pos_v, b0, b1, b2, isem, gs0, gs1, gs2, ws0, ws1, ws2):
    bufs = [b0, b1, b2]
    gsems = [gs0, gs1, gs2]
    wsems = [ws0, ws1, ws2]
    wid = lax.axis_index("s") * NUM_CORES + lax.axis_index("c")
    pbase = wid * P_PER_W

    idx_hs = [
        pltpu.async_copy(idx_hbm.at[b, pl.ds(pbase, P_PER_W)],
                         idx_v.at[pl.ds(b * P_PER_W, P_PER_W)], isem)
        for b in range(BATCH)
    ]
    pos_h = pltpu.async_copy(
        pos_hbm.at[pl.ds(pbase * DIM, P_PER_W * DIM)], pos_v, isem)
    for h in idx_hs[:B_PER_STEP]:
        h.wait()

    def start_gather(s):
        idx_slice = idx_v.at[pl.ds(s * ROWS_PER_STEP, ROWS_PER_STEP)]
        return pltpu.async_copy(table_hbm.at[idx_slice], bufs[s % NBUF],
                                gsems[s % NBUF])

    gather_h = {0: start_gather(0)}
    for h in idx_hs[B_PER_STEP:]:
        h.wait()
    gather_h[1] = start_gather(1)
    pos_h.wait()

    write_h = {}
    for s in range(N_STEPS):
        buf = bufs[s % NBUF]
        gather_h.pop(s).wait()
        if s + 2 < N_STEPS:
            for h in write_h.pop(s + 2 - NBUF, ()):
                h.wait()
            gather_h[s + 2] = start_gather(s + 2)
        for h in write_h.pop(s - NBUF, ()):
            h.wait()

        cur = buf

        @plsc.parallel_loop(0, P_PER_W, 1)
        def fma_pos(p):
            for j in range(VECS_PER_ROW):
                pv = pos_v[pl.ds(p * DIM + j * LANES, LANES)]
                for bb in range(B_PER_STEP):
                    r = bb * P_PER_W + p
                    cur[r, pl.ds(j * LANES, LANES)] = (
                        cur[r, pl.ds(j * LANES, LANES)] * SCALE + pv)

        hs = []
        for bb in range(B_PER_STEP):
            b = s * B_PER_STEP + bb
            hs.append(pltpu.async_copy(
                buf.at[pl.ds(bb * P_PER_W, P_PER_W)],
                out_hbm.at[b, pl.ds(pbase, P_PER_W)],
                wsems[s % NBUF]))
        write_h[s] = hs

    for hs in write_h.values():
        for h in hs:
            h.wait()


@jax.jit
def _embed(idx, table):
    pos_enc = jnp.asarray(_POS_ENC)
    mesh = plsc.VectorSubcoreMesh(
        core_axis_name="c", subcore_axis_name="s",
        num_cores=NUM_CORES, num_subcores=NUM_SUBCORES)
    fn = pl.kernel(
        _embed_body,
        out_type=jax.ShapeDtypeStruct((BATCH, SEQ_LEN, DIM), jnp.float32),
        mesh=mesh,
        scratch_types=[
            pltpu.VMEM((BATCH * P_PER_W,), jnp.int32),
            pltpu.VMEM((P_PER_W * DIM,), jnp.float32),
            pltpu.VMEM((ROWS_PER_STEP, DIM), jnp.float32),
            pltpu.VMEM((ROWS_PER_STEP, DIM), jnp.float32),
            pltpu.VMEM((ROWS_PER_STEP, DIM), jnp.float32),
            pltpu.SemaphoreType.DMA,
            pltpu.SemaphoreType.DMA,
            pltpu.SemaphoreType.DMA,
            pltpu.SemaphoreType.DMA,
            pltpu.SemaphoreType.DMA,
            pltpu.SemaphoreType.DMA,
            pltpu.SemaphoreType.DMA,
        ],
    )
    return fn(table, idx, pos_enc)


def kernel(inputs, table):
    return _embed(inputs.astype(jnp.int32), table)
